# baseline (speedup 1.0000x reference)
import jax
import jax.numpy as jnp
from jax.experimental import pallas as pl
from jax.experimental.pallas import tpu as pltpu

_BK = 512


def _kmajor_kernel(x_ref, w_ref, b_ref, adj_ref, out_ref):
    din = w_ref.shape[1]
    x = x_ref[...]
    h0 = jnp.dot(x[:, :din], w_ref[0], preferred_element_type=jnp.float32) + b_ref[0]
    h1 = jnp.dot(x[:, din:], w_ref[1], preferred_element_type=jnp.float32) + b_ref[1]
    part = (
        jnp.dot(adj_ref[0], h0, preferred_element_type=jnp.float32)
        + jnp.dot(adj_ref[1], h1, preferred_element_type=jnp.float32)
    )

    @pl.when(pl.program_id(0) == 0)
    def _():
        out_ref[...] = part

    @pl.when(pl.program_id(0) != 0)
    def _():
        out_ref[...] += part


def kernel(x, adjs, W1, b1, W2, b2):
    n = adjs.shape[1]
    dout = W1.shape[1]
    w = jnp.stack([W1, W2])
    b = jnp.stack([b1, b2]).reshape(2, 1, dout)

    out = pl.pallas_call(
        _kmajor_kernel,
        grid=(n // _BK,),
        in_specs=[
            pl.BlockSpec((_BK, x.shape[1]), lambda k: (k, 0)),
            pl.BlockSpec((2, W1.shape[0], dout), lambda k: (0, 0, 0)),
            pl.BlockSpec((2, 1, dout), lambda k: (0, 0, 0)),
            pl.BlockSpec((2, n, _BK), lambda k: (0, 0, k)),
        ],
        out_specs=pl.BlockSpec((n, dout), lambda k: (0, 0)),
        out_shape=jax.ShapeDtypeStruct((n, dout), jnp.float32),
        compiler_params=pltpu.CompilerParams(dimension_semantics=("arbitrary",)),
    )(x, w, b, adjs)
    return out


# speedup vs baseline: 1.0302x; 1.0302x over previous
import jax
import jax.numpy as jnp
from jax.experimental import pallas as pl
from jax.experimental.pallas import tpu as pltpu

_BM = 256


def _block_copy(adj_hbm, abuf, sem, blk, slot):
    return pltpu.make_async_copy(
        adj_hbm.at[:, pl.ds(blk * _BM, _BM), :], abuf.at[slot], sem.at[slot]
    )


def _manual_kernel(x_ref, w_ref, b_ref, adj_hbm, out_ref, h_ref, abuf, sem):
    i = pl.program_id(0)
    nb = pl.num_programs(0)

    @pl.when(i == 0)
    def _():
        _block_copy(adj_hbm, abuf, sem, 0, 0).start()
        din = w_ref.shape[1]
        x = x_ref[...]
        h_ref[0] = jnp.dot(x[:, :din], w_ref[0], preferred_element_type=jnp.float32) + b_ref[0]
        h_ref[1] = jnp.dot(x[:, din:], w_ref[1], preferred_element_type=jnp.float32) + b_ref[1]

    @pl.when(i + 1 < nb)
    def _():
        _block_copy(adj_hbm, abuf, sem, i + 1, (i + 1) % 2).start()

    _block_copy(adj_hbm, abuf, sem, i, i % 2).wait()
    a = abuf[i % 2]
    out_ref[...] = (
        jnp.dot(a[0], h_ref[0], preferred_element_type=jnp.float32)
        + jnp.dot(a[1], h_ref[1], preferred_element_type=jnp.float32)
    )


def kernel(x, adjs, W1, b1, W2, b2):
    n = adjs.shape[1]
    dout = W1.shape[1]
    w = jnp.stack([W1, W2])
    b = jnp.stack([b1, b2]).reshape(2, 1, dout)

    out = pl.pallas_call(
        _manual_kernel,
        grid=(n // _BM,),
        in_specs=[
            pl.BlockSpec((n, x.shape[1]), lambda i: (0, 0)),
            pl.BlockSpec((2, W1.shape[0], dout), lambda i: (0, 0, 0)),
            pl.BlockSpec((2, 1, dout), lambda i: (0, 0, 0)),
            pl.BlockSpec(memory_space=pl.ANY),
        ],
        out_specs=pl.BlockSpec((_BM, dout), lambda i: (i, 0)),
        out_shape=jax.ShapeDtypeStruct((n, dout), jnp.float32),
        scratch_shapes=[
            pltpu.VMEM((2, n, dout), jnp.float32),
            pltpu.VMEM((2, 2, _BM, n), jnp.float32),
            pltpu.SemaphoreType.DMA((2,)),
        ],
        compiler_params=pltpu.CompilerParams(dimension_semantics=("arbitrary",)),
    )(x, w, b, adjs)
    return out


# speedup vs baseline: 1.0344x; 1.0041x over previous
import jax
import jax.numpy as jnp
from jax.experimental import pallas as pl
from jax.experimental.pallas import tpu as pltpu

_BM = 256


def _plane_copy(adj_hbm, abuf, sem, blk, slot, plane):
    return pltpu.make_async_copy(
        adj_hbm.at[plane, pl.ds(blk * _BM, _BM), :],
        abuf.at[slot, plane],
        sem.at[slot, plane],
    )


def _start_block(adj_hbm, abuf, sem, blk, slot):
    _plane_copy(adj_hbm, abuf, sem, blk, slot, 0).start()
    _plane_copy(adj_hbm, abuf, sem, blk, slot, 1).start()


def _wait_block(adj_hbm, abuf, sem, blk, slot):
    _plane_copy(adj_hbm, abuf, sem, blk, slot, 0).wait()
    _plane_copy(adj_hbm, abuf, sem, blk, slot, 1).wait()


def _manual_kernel(x_ref, w_ref, b_ref, adj_hbm, out_ref, h_ref, abuf, sem):
    i = pl.program_id(0)
    nb = pl.num_programs(0)

    @pl.when(i == 0)
    def _():
        _start_block(adj_hbm, abuf, sem, 0, 0)
        din = w_ref.shape[1]
        x = x_ref[...]
        h_ref[0] = jnp.dot(x[:, :din], w_ref[0], preferred_element_type=jnp.float32) + b_ref[0]
        h_ref[1] = jnp.dot(x[:, din:], w_ref[1], preferred_element_type=jnp.float32) + b_ref[1]

    @pl.when(i + 1 < nb)
    def _():
        _start_block(adj_hbm, abuf, sem, i + 1, (i + 1) % 2)

    _wait_block(adj_hbm, abuf, sem, i, i % 2)
    a = abuf[i % 2]
    out_ref[...] = (
        jnp.dot(a[0], h_ref[0], preferred_element_type=jnp.float32)
        + jnp.dot(a[1], h_ref[1], preferred_element_type=jnp.float32)
    )


def kernel(x, adjs, W1, b1, W2, b2):
    n = adjs.shape[1]
    dout = W1.shape[1]
    w = jnp.stack([W1, W2])
    b = jnp.stack([b1, b2]).reshape(2, 1, dout)

    out = pl.pallas_call(
        _manual_kernel,
        grid=(n // _BM,),
        in_specs=[
            pl.BlockSpec((n, x.shape[1]), lambda i: (0, 0)),
            pl.BlockSpec((2, W1.shape[0], dout), lambda i: (0, 0, 0)),
            pl.BlockSpec((2, 1, dout), lambda i: (0, 0, 0)),
            pl.BlockSpec(memory_space=pl.ANY),
        ],
        out_specs=pl.BlockSpec((_BM, dout), lambda i: (i, 0)),
        out_shape=jax.ShapeDtypeStruct((n, dout), jnp.float32),
        scratch_shapes=[
            pltpu.VMEM((2, n, dout), jnp.float32),
            pltpu.VMEM((2, 2, _BM, n), jnp.float32),
            pltpu.SemaphoreType.DMA((2, 2)),
        ],
        compiler_params=pltpu.CompilerParams(dimension_semantics=("arbitrary",)),
    )(x, w, b, adjs)
    return out
